# baseline (speedup 1.0000x reference)
import jax
import jax.numpy as jnp
from jax.experimental import pallas as pl


def _rowdot_body(gu_ref, gi_ref, out_ref):
    out_ref[:, :] = jnp.sum(gu_ref[:, :] * gi_ref[:, :], axis=1, keepdims=True)


def kernel(gu, gi):
    B, D = gu.shape
    blk = 2048
    xui = pl.pallas_call(
        _rowdot_body,
        grid=(B // blk,),
        in_specs=[
            pl.BlockSpec((blk, D), lambda i: (i, 0)),
            pl.BlockSpec((blk, D), lambda i: (i, 0)),
        ],
        out_specs=pl.BlockSpec((blk, 1), lambda i: (i, 0)),
        out_shape=jax.ShapeDtypeStruct((B, 1), jnp.float32),
    )(gu, gi)
    return (xui[:, 0], gu, gi)


# speedup vs baseline: 1.2683x; 1.2683x over previous
import jax
import jax.numpy as jnp
from jax.experimental import pallas as pl


def _body(gu_ref, gi_ref, xui_ref, gu_out_ref, gi_out_ref):
    u = gu_ref[:, :]
    v = gi_ref[:, :]
    gu_out_ref[:, :] = u
    gi_out_ref[:, :] = v
    ones = jnp.ones((u.shape[1], 1), dtype=jnp.float32)
    xui_ref[:, :] = jax.lax.dot_general(
        u * v, ones, (((1,), (0,)), ((), ())),
        preferred_element_type=jnp.float32)


def kernel(gu, gi):
    B, D = gu.shape
    blk = 2048
    xui, gu_o, gi_o = pl.pallas_call(
        _body,
        grid=(B // blk,),
        in_specs=[
            pl.BlockSpec((blk, D), lambda i: (i, 0)),
            pl.BlockSpec((blk, D), lambda i: (i, 0)),
        ],
        out_specs=[
            pl.BlockSpec((blk, 1), lambda i: (i, 0)),
            pl.BlockSpec((blk, D), lambda i: (i, 0)),
            pl.BlockSpec((blk, D), lambda i: (i, 0)),
        ],
        out_shape=[
            jax.ShapeDtypeStruct((B, 1), jnp.float32),
            jax.ShapeDtypeStruct((B, D), jnp.float32),
            jax.ShapeDtypeStruct((B, D), jnp.float32),
        ],
    )(gu, gi)
    return (xui[:, 0], gu_o, gi_o)


# speedup vs baseline: 1.3568x; 1.0698x over previous
import jax
import jax.numpy as jnp
from jax.experimental import pallas as pl


def _body(gu_ref, gi_ref, xui_ref, gu_out_ref, gi_out_ref):
    u = gu_ref[:, :]
    v = gi_ref[:, :]
    gu_out_ref[:, :] = u
    gi_out_ref[:, :] = v
    ones = jnp.ones((u.shape[1], 1), dtype=jnp.float32)
    xui_ref[:, :] = jax.lax.dot_general(
        u * v, ones, (((1,), (0,)), ((), ())),
        preferred_element_type=jnp.float32)


def kernel(gu, gi):
    B, D = gu.shape
    blk = 4096
    xui, gu_o, gi_o = pl.pallas_call(
        _body,
        grid=(B // blk,),
        in_specs=[
            pl.BlockSpec((blk, D), lambda i: (i, 0)),
            pl.BlockSpec((blk, D), lambda i: (i, 0)),
        ],
        out_specs=[
            pl.BlockSpec((blk, 1), lambda i: (i, 0)),
            pl.BlockSpec((blk, D), lambda i: (i, 0)),
            pl.BlockSpec((blk, D), lambda i: (i, 0)),
        ],
        out_shape=[
            jax.ShapeDtypeStruct((B, 1), jnp.float32),
            jax.ShapeDtypeStruct((B, D), jnp.float32),
            jax.ShapeDtypeStruct((B, D), jnp.float32),
        ],
    )(gu, gi)
    return (xui[:, 0], gu_o, gi_o)


# speedup vs baseline: 1.4021x; 1.0334x over previous
import jax
import jax.numpy as jnp
from jax.experimental import pallas as pl


def _body(gu_ref, gi_ref, xui_ref, gu_out_ref, gi_out_ref):
    u = gu_ref[:, :]
    v = gi_ref[:, :]
    gu_out_ref[:, :] = u
    gi_out_ref[:, :] = v
    ones = jnp.ones((u.shape[1], 1), dtype=jnp.float32)
    xui_ref[:, :] = jax.lax.dot_general(
        u * v, ones, (((1,), (0,)), ((), ())),
        preferred_element_type=jnp.float32)


def kernel(gu, gi):
    B, D = gu.shape
    blk = 8192
    xui, gu_o, gi_o = pl.pallas_call(
        _body,
        grid=(B // blk,),
        in_specs=[
            pl.BlockSpec((blk, D), lambda i: (i, 0)),
            pl.BlockSpec((blk, D), lambda i: (i, 0)),
        ],
        out_specs=[
            pl.BlockSpec((blk, 1), lambda i: (i, 0)),
            pl.BlockSpec((blk, D), lambda i: (i, 0)),
            pl.BlockSpec((blk, D), lambda i: (i, 0)),
        ],
        out_shape=[
            jax.ShapeDtypeStruct((B, 1), jnp.float32),
            jax.ShapeDtypeStruct((B, D), jnp.float32),
            jax.ShapeDtypeStruct((B, D), jnp.float32),
        ],
    )(gu, gi)
    return (xui[:, 0], gu_o, gi_o)
